# baseline (speedup 1.0000x reference)
import dataclasses
import functools

import jax
import jax.numpy as jnp
from jax import lax
from jax.experimental import pallas as pl
from jax.experimental.pallas import tpu as pltpu
from jax.experimental.pallas import tpu_sc as plsc

B = 4096
D = 32
NUM_WORKERS = 32
B_PER_W = B // NUM_WORKERS
LANES = 16


def _sc_gather_dot(word_u, word_v, in_embed, out_embed, in_bias_flat,
                   out_bias_flat):
  mesh = plsc.VectorSubcoreMesh(core_axis_name="c", subcore_axis_name="s")
  cp = pltpu.CompilerParams()
  if "needs_layout_passes" in pltpu.CompilerParams.__dataclass_fields__:
    cp = dataclasses.replace(cp, needs_layout_passes=False)
  if "use_tc_tiling_on_sc" in pltpu.CompilerParams.__dataclass_fields__:
    cp = dataclasses.replace(cp, use_tc_tiling_on_sc=False)

  @functools.partial(
      pl.kernel,
      compiler_params=cp,
      out_type=(
          jax.ShapeDtypeStruct((B,), jnp.float32),
          jax.ShapeDtypeStruct((B,), jnp.float32),
      ),
      mesh=mesh,
      scratch_types=[
          pltpu.VMEM((B_PER_W,), jnp.int32),
          pltpu.VMEM((B_PER_W,), jnp.int32),
          pltpu.VMEM((B_PER_W, D), jnp.float32),
          pltpu.VMEM((B_PER_W, D), jnp.float32),
          pltpu.VMEM((B_PER_W,), jnp.float32),
          pltpu.VMEM((B_PER_W,), jnp.float32),
          pltpu.VMEM((B_PER_W,), jnp.float32),
          pltpu.VMEM((B_PER_W,), jnp.float32),
          pltpu.SemaphoreType.DMA,
      ],
  )
  def k(word_u_hbm, word_v_hbm, in_embed_hbm, out_embed_hbm, in_bias_hbm,
        out_bias_hbm, dots_hbm, colsum_hbm, idx_u, idx_v, u_rows, v_rows,
        ub, vb, dots_v, colsum_v, sem):
    wid = lax.axis_index("s") * 2 + lax.axis_index("c")
    base = wid * B_PER_W

    c1 = pltpu.async_copy(word_u_hbm.at[pl.ds(base, B_PER_W)], idx_u, sem)
    c2 = pltpu.async_copy(word_v_hbm.at[pl.ds(base, B_PER_W)], idx_v, sem)
    c1.wait()
    c2.wait()

    g1 = pltpu.async_copy(in_embed_hbm.at[idx_u], u_rows, sem)
    g2 = pltpu.async_copy(out_embed_hbm.at[idx_v], v_rows, sem)
    g3 = pltpu.async_copy(in_bias_hbm.at[idx_u], ub, sem)
    g4 = pltpu.async_copy(out_bias_hbm.at[idx_v], vb, sem)
    g1.wait()
    g2.wait()
    g3.wait()
    g4.wait()

    iota = lax.iota(jnp.int32, LANES)
    for g in range(B_PER_W // LANES):
      rows = iota + (g * LANES)
      acc = jnp.zeros((LANES,), jnp.float32)
      for d in range(D):
        cols = jnp.full((LANES,), d, jnp.int32)
        uu = plsc.load_gather(u_rows, [rows, cols])
        vv = plsc.load_gather(v_rows, [rows, cols])
        acc = acc + uu * vv
      dots_v[pl.ds(g * LANES, LANES)] = acc
      colsum_v[pl.ds(g * LANES, LANES)] = (
          ub[pl.ds(g * LANES, LANES)] + vb[pl.ds(g * LANES, LANES)])

    s1 = pltpu.async_copy(dots_v, dots_hbm.at[pl.ds(base, B_PER_W)], sem)
    s2 = pltpu.async_copy(colsum_v, colsum_hbm.at[pl.ds(base, B_PER_W)], sem)
    s1.wait()
    s2.wait()

  return k(word_u, word_v, in_embed, out_embed, in_bias_flat, out_bias_flat)


_COL_BLK = 512


def _tc_broadcast_body(colsum_ref, dots_ref, out_ref):
  out_ref[...] = colsum_ref[...] + dots_ref[...]


def _tc_broadcast(colsum_col, dots_row):
  return pl.pallas_call(
      _tc_broadcast_body,
      grid=(B // _COL_BLK,),
      in_specs=[
          pl.BlockSpec((B, 1), lambda j: (0, 0)),
          pl.BlockSpec((1, _COL_BLK), lambda j: (0, j)),
      ],
      out_specs=pl.BlockSpec((B, _COL_BLK), lambda j: (0, j)),
      out_shape=jax.ShapeDtypeStruct((B, B), jnp.float32),
  )(colsum_col, dots_row)


def kernel(word_u, word_v, in_embed, in_bias, out_embed, out_bias):
  dots, colsum = _sc_gather_dot(
      word_u.astype(jnp.int32), word_v.astype(jnp.int32),
      in_embed, out_embed,
      in_bias.reshape(-1), out_bias.reshape(-1))
  return _tc_broadcast(colsum.reshape(B, 1), dots.reshape(1, B))


# speedup vs baseline: 4.7813x; 4.7813x over previous
import dataclasses
import functools

import jax
import jax.numpy as jnp
from jax import lax
from jax.experimental import pallas as pl
from jax.experimental.pallas import tpu as pltpu
from jax.experimental.pallas import tpu_sc as plsc

B = 4096
D = 32
NUM_WORKERS = 32
B_PER_W = B // NUM_WORKERS
LANES = 16
W = 128
CH = 16
N_CH = B_PER_W // CH


def _sc_gather_dot(word_u, word_v, in_embed_t, out_embed_t, in_bias_flat,
                   out_bias_flat):
  mesh = plsc.VectorSubcoreMesh(core_axis_name="c", subcore_axis_name="s")
  cp = pltpu.CompilerParams()
  if "needs_layout_passes" in pltpu.CompilerParams.__dataclass_fields__:
    cp = dataclasses.replace(cp, needs_layout_passes=False)
  if "use_tc_tiling_on_sc" in pltpu.CompilerParams.__dataclass_fields__:
    cp = dataclasses.replace(cp, use_tc_tiling_on_sc=True)

  @functools.partial(
      pl.kernel,
      compiler_params=cp,
      out_type=(
          jax.ShapeDtypeStruct((B,), jnp.float32),
          jax.ShapeDtypeStruct((B,), jnp.float32),
      ),
      mesh=mesh,
      scratch_types=[
          pltpu.VMEM((B_PER_W,), jnp.int32),
          pltpu.VMEM((B_PER_W,), jnp.int32),
          pltpu.VMEM((CH, D, W), jnp.float32),
          pltpu.VMEM((D, CH), jnp.float32),
          pltpu.VMEM((B_PER_W,), jnp.float32),
          pltpu.VMEM((B_PER_W,), jnp.float32),
          pltpu.VMEM((B_PER_W,), jnp.float32),
          pltpu.VMEM((B_PER_W,), jnp.float32),
          pltpu.SemaphoreType.DMA,
          pltpu.SemaphoreType.DMA,
      ],
  )
  def k(word_u_hbm, word_v_hbm, u_tab_hbm, v_tab_hbm, in_bias_hbm,
        out_bias_hbm, dots_hbm, colsum_hbm, idx_u_v,
        idx_v_v, blk, u_comp, ub, vb, dots_v, colsum_v, sem, bsem):
    wid = lax.axis_index("s") * 2 + lax.axis_index("c")
    base = wid * B_PER_W

    c1 = pltpu.async_copy(word_u_hbm.at[pl.ds(base, B_PER_W)], idx_u_v, bsem)
    c2 = pltpu.async_copy(word_v_hbm.at[pl.ds(base, B_PER_W)], idx_v_v, bsem)
    c1.wait()
    c2.wait()
    g3 = pltpu.async_copy(in_bias_hbm.at[idx_u_v], ub, bsem)
    g4 = pltpu.async_copy(out_bias_hbm.at[idx_v_v], vb, bsem)

    iota = lax.iota(jnp.int32, LANES)
    for c in range(N_CH):
      off = c * CH
      lu = idx_u_v[pl.ds(off, LANES)]
      lv = idx_v_v[pl.ds(off, LANES)]
      jb_u = (lu // W) * W
      jb_v = (lv // W) * W
      cops = []
      for i in range(CH):
        ju = jnp.sum(jnp.where(iota == i, jb_u, 0))
        cops.append(pltpu.async_copy(
            u_tab_hbm.at[:, pl.ds(pl.multiple_of(ju, W), W)], blk.at[i], sem))
      for cp_ in cops:
        cp_.wait()
      lane_u = lu - jb_u
      for d in range(D):
        dd = jnp.full((LANES,), d, jnp.int32)
        u_comp[d, :] = plsc.load_gather(blk, [iota, dd, lane_u])
      cops = []
      for i in range(CH):
        jv = jnp.sum(jnp.where(iota == i, jb_v, 0))
        cops.append(pltpu.async_copy(
            v_tab_hbm.at[:, pl.ds(pl.multiple_of(jv, W), W)], blk.at[i], sem))
      for cp_ in cops:
        cp_.wait()
      lane_v = lv - jb_v
      acc = jnp.zeros((LANES,), jnp.float32)
      for d in range(D):
        dd = jnp.full((LANES,), d, jnp.int32)
        vv = plsc.load_gather(blk, [iota, dd, lane_v])
        acc = acc + u_comp[d, :] * vv
      dots_v[pl.ds(off, LANES)] = acc

    g3.wait()
    g4.wait()
    for g in range(B_PER_W // LANES):
      colsum_v[pl.ds(g * LANES, LANES)] = (
          ub[pl.ds(g * LANES, LANES)] + vb[pl.ds(g * LANES, LANES)])

    s1 = pltpu.async_copy(dots_v, dots_hbm.at[pl.ds(base, B_PER_W)], bsem)
    s2 = pltpu.async_copy(colsum_v, colsum_hbm.at[pl.ds(base, B_PER_W)], bsem)
    s1.wait()
    s2.wait()

  return k(word_u, word_v, in_embed_t, out_embed_t, in_bias_flat,
           out_bias_flat)


_COL_BLK = 512


def _tc_broadcast_body(colsum_ref, dots_ref, out_ref):
  out_ref[...] = colsum_ref[...] + dots_ref[...]


def _tc_broadcast(colsum_col, dots_row):
  return pl.pallas_call(
      _tc_broadcast_body,
      grid=(B // _COL_BLK,),
      in_specs=[
          pl.BlockSpec((B, 1), lambda j: (0, 0)),
          pl.BlockSpec((1, _COL_BLK), lambda j: (0, j)),
      ],
      out_specs=pl.BlockSpec((B, _COL_BLK), lambda j: (0, j)),
      out_shape=jax.ShapeDtypeStruct((B, B), jnp.float32),
  )(colsum_col, dots_row)


def kernel(word_u, word_v, in_embed, in_bias, out_embed, out_bias):
  dots, colsum = _sc_gather_dot(
      word_u.astype(jnp.int32), word_v.astype(jnp.int32),
      in_embed.T, out_embed.T,
      in_bias.reshape(-1), out_bias.reshape(-1))
  return _tc_broadcast(colsum.reshape(B, 1), dots.reshape(1, B))
